# baseline (speedup 1.0000x reference)
import functools

import jax
import jax.numpy as jnp
import numpy as np
from jax import lax
from jax.experimental import pallas as pl
from jax.experimental.pallas import tpu as pltpu
from jax.experimental.pallas import tpu_sc as plsc

H, W, C, M1, M2, KD, ED = 64, 64, 16, 12, 12, 64, 4
N = 4 * H * W
E = 262144
NW = 32
PER_W = E // NW
CH = 2048
NCH = PER_W // CH
IDX_ROWS = CH // 128


def _gelu(x):
    return 0.5 * x * (1.0 + lax.erf(x * np.float32(1.0 / np.sqrt(2.0))))


def _inorm_rows(x_ref, o_ref, b):
    for i in range(b):
        xb = x_ref[pl.ds(i * H * W, H * W), :]
        m = jnp.mean(xb, axis=0, keepdims=True)
        v = jnp.mean((xb - m) * (xb - m), axis=0, keepdims=True)
        o_ref[pl.ds(i * H * W, H * W), :] = (xb - m) * lax.rsqrt(v + 1e-5)


def _pre_body(nodes_ref, fcw1_ref, fcb1_ref, fcw2_ref, fcb2_ref, root0_ref,
              bias0_ref, xn_ref, x3_ref, r0_ref):
    _inorm_rows(nodes_ref, xn_ref, 4)
    nd = nodes_ref[...]
    h = _gelu(jnp.dot(nd, fcw1_ref[...], preferred_element_type=jnp.float32)
              + fcb1_ref[...])
    x3_ref[...] = jnp.dot(h, fcw2_ref[...], preferred_element_type=jnp.float32) \
        + fcb2_ref[...]
    r0_ref[...] = jnp.dot(nd, root0_ref[...], preferred_element_type=jnp.float32) \
        + bias0_ref[...]


def _specw_body(x_ref, gwr_ref, gwi_ref, yr_ref, yi_ref):
    x = x_ref[...]
    yr_ref[...] = jnp.dot(x, gwr_ref[...], preferred_element_type=jnp.float32)
    yi_ref[...] = jnp.dot(x, gwi_ref[...], preferred_element_type=jnp.float32)


def _spech_body(yr_ref, yi_ref, fhr_ref, fhi_ref, ar_ref, ai_ref):
    yr = yr_ref[...]
    yi = yi_ref[...]
    fhr = fhr_ref[...]
    fhi = fhi_ref[...]
    ar_ref[...] = jnp.dot(fhr, yr, preferred_element_type=jnp.float32) \
        - jnp.dot(fhi, yi, preferred_element_type=jnp.float32)
    ai_ref[...] = jnp.dot(fhr, yi, preferred_element_type=jnp.float32) \
        + jnp.dot(fhi, yr, preferred_element_type=jnp.float32)


def _specmix_body(ar_ref, ai_ref, wtr_ref, wti_ref, osr_ref, osi_ref):
    osr = jnp.zeros((24, 4, 16, 12), jnp.float32)
    osi = jnp.zeros((24, 4, 16, 12), jnp.float32)
    yvr = ar_ref[...]
    yvi = ai_ref[...]
    for i in range(16):
        br = yvr[:, :, i, :][:, :, None, :]
        bi = yvi[:, :, i, :][:, :, None, :]
        wr = wtr_ref[i][:, None, :, :]
        wi = wti_ref[i][:, None, :, :]
        osr = osr + br * wr - bi * wi
        osi = osi + br * wi + bi * wr
    osr_ref[...] = osr
    osi_ref[...] = osi


def _specih_body(osr_ref, osi_ref, ehr_ref, ehi_ref, pr_ref, pi_ref):
    osr = osr_ref[...]
    osi = osi_ref[...]
    ehr = ehr_ref[...]
    ehi = ehi_ref[...]
    pr_ref[...] = jnp.dot(ehr, osr, preferred_element_type=jnp.float32) \
        - jnp.dot(ehi, osi, preferred_element_type=jnp.float32)
    pi_ref[...] = jnp.dot(ehr, osi, preferred_element_type=jnp.float32) \
        + jnp.dot(ehi, osr, preferred_element_type=jnp.float32)


def _speciw_body(pr_ref, pi_ref, dwr_ref, dwi_ref, z_ref):
    z_ref[...] = jnp.dot(pr_ref[...], dwr_ref[...],
                         preferred_element_type=jnp.float32) \
        - jnp.dot(pi_ref[...], dwi_ref[...],
                  preferred_element_type=jnp.float32)


def _post_body(z_ref, c1w_ref, c1b_ref, c2w_ref, c2b_ref, x1_ref, zn_ref):
    _inorm_rows(z_ref, zn_ref, 4)
    h = _gelu(jnp.dot(zn_ref[...], c1w_ref[...],
                      preferred_element_type=jnp.float32) + c1b_ref[...])
    x1_ref[...] = jnp.dot(h, c2w_ref[...], preferred_element_type=jnp.float32) \
        + c2b_ref[...]


def _mlp_body(ea_ref, kw1_ref, kb1_ref, kw2_ref, kb2_ref, h2_ref):
    h1 = _gelu(jnp.dot(ea_ref[...], kw1_ref[...],
                       preferred_element_type=jnp.float32) + kb1_ref[...])
    h2_ref[...] = _gelu(jnp.dot(h1, kw2_ref[...],
                                preferred_element_type=jnp.float32)
                        + kb2_ref[...])


def _msg_body(h2_ref, g_ref, kw3_ref, kb3_ref, rm_ref, sm_ref, msg_ref):
    wf = jnp.dot(h2_ref[...], kw3_ref[...],
                 preferred_element_type=jnp.float32) + kb3_ref[...]
    xe = jnp.dot(g_ref[...], rm_ref[...], preferred_element_type=jnp.float32)
    msg_ref[...] = jnp.dot(xe * wf, sm_ref[...],
                           preferred_element_type=jnp.float32)


def _combine_body(parts_ref, root1_ref, bias1_ref, g1_ref, r1_ref):
    g1 = _gelu(parts_ref[0] + parts_ref[1])
    g1_ref[...] = g1
    r1_ref[...] = jnp.dot(g1, root1_ref[...],
                          preferred_element_type=jnp.float32) + bias1_ref[...]


def _final_body(x1_ref, parts_ref, x3_ref, out_ref):
    out_ref[...] = _gelu(x1_ref[...] + parts_ref[0] + parts_ref[1] + x3_ref[...])


def _tc_call(body, out_shapes, *args):
    return pl.pallas_call(
        body,
        out_shape=out_shapes,
    )(*args)


def _sc_gather(g, src):
    mesh = plsc.VectorSubcoreMesh(core_axis_name="c", subcore_axis_name="s")

    @functools.partial(
        pl.kernel,
        out_type=jax.ShapeDtypeStruct((E, 16), jnp.float32),
        mesh=mesh,
        compiler_params=pltpu.CompilerParams(use_tc_tiling_on_sc=False),
        scratch_types=[
            pltpu.VMEM((CH,), jnp.int32),
            pltpu.VMEM((CH, 16), jnp.float32),
            pltpu.SemaphoreType.DMA,
        ],
    )
    def k(g_hbm, src_hbm, out_hbm, idx_v, rows_v, sem):
        wid = lax.axis_index("s") * 2 + lax.axis_index("c")
        for j in range(NCH):
            base = wid * PER_W + j * CH
            pltpu.sync_copy(src_hbm.at[pl.ds(base, CH)], idx_v)
            pltpu.async_copy(g_hbm.at[idx_v], rows_v, sem).wait()
            pltpu.sync_copy(rows_v, out_hbm.at[pl.ds(base, CH)])

    return k(g, src).reshape(E // 8, 128)


def _sc_scatter(msg, dst2d, seed):
    mesh = plsc.VectorSubcoreMesh(core_axis_name="c", subcore_axis_name="s")

    @functools.partial(
        pl.kernel,
        out_type=jax.ShapeDtypeStruct((2, N, 16), jnp.float32),
        mesh=mesh,
        compiler_params=pltpu.CompilerParams(use_tc_tiling_on_sc=False),
        scratch_types=[
            pltpu.VMEM((IDX_ROWS, 128), jnp.int32),
            pltpu.VMEM((CH, 16), jnp.float32),
            pltpu.VMEM_SHARED((N, 16), jnp.float32),
        ],
    )
    def k(msg_hbm, dst_hbm, seed_hbm, out_hbm, idx_v, msg_v, acc_sh):
        cid = lax.axis_index("c")
        sid = lax.axis_index("s")
        wid = sid * 2 + cid

        @pl.when(sid == 0)
        def _():
            pltpu.sync_copy(seed_hbm.at[cid], acc_sh)

        plsc.subcore_barrier()
        for j in range(NCH):
            ebase = wid * PER_W + j * CH
            pltpu.sync_copy(dst_hbm.at[pl.ds(ebase // 128, IDX_ROWS)], idx_v)
            pltpu.sync_copy(msg_hbm.at[pl.ds(ebase, CH)], msg_v)
            for t in range(IDX_ROWS):
                pltpu.sync_copy(msg_v.at[pl.ds(t * 128, 128)],
                                acc_sh.at[idx_v.at[t]], add=True)
        plsc.subcore_barrier()
        rows = N // 16
        pltpu.sync_copy(acc_sh.at[pl.ds(sid * rows, rows)],
                        out_hbm.at[cid, pl.ds(sid * rows, rows)])

    return k(msg, dst2d, seed)


def kernel(nodes, edge_index, edge_attrs, batchsize, w1r, w1i, w2r, w2i,
           conv1_w, conv1_b, conv2_w, conv2_b, k_w1, k_b1, k_w2, k_b2,
           k_w3, k_b3, root0, bias0, root1, bias1, fc_w1, fc_b1, fc_w2,
           fc_b2):
    f32 = jnp.float32
    src = edge_index[0]
    dst2d = edge_index[1].reshape(E // 128, 128)

    h_idx = np.arange(H)
    r_fwd = np.concatenate([np.arange(M1), np.arange(H - M1, H)])
    fh = np.exp(-2j * np.pi * np.outer(r_fwd, h_idx) / H) / H
    gw = np.exp(-2j * np.pi * np.outer(np.arange(W), np.arange(M2)) / W)
    rho = np.concatenate([np.arange(M1), np.arange(21, 33)])
    eh = np.exp(2j * np.pi * np.outer(h_idx, rho) / H)
    alpha = np.where(np.arange(M2) == 0, 1.0, 2.0)
    dw = alpha[:, None] * np.exp(
        2j * np.pi * np.outer(np.arange(M2), np.arange(W)) / W) / W
    consts = [jnp.asarray(m, f32) for m in
              (fh.real, fh.imag, gw.real, gw.imag)]
    wtr = jnp.concatenate([w1r, w2r], axis=2).transpose(0, 2, 1, 3)
    wti = jnp.concatenate([w1i, w2i], axis=2).transpose(0, 2, 1, 3)
    iconsts = [jnp.asarray(m, f32) for m in
               (eh.real, eh.imag, dw.real, dw.imag)]

    i8 = jnp.eye(8, dtype=f32)
    rm8 = jnp.asarray(np.kron(np.eye(8),
                              np.kron(np.eye(16), np.ones((1, 16)))), f32)
    sm8 = jnp.asarray(np.kron(np.eye(8), np.tile(np.eye(16), (16, 1))), f32)
    kw1p = jnp.kron(i8, k_w1)
    kw2p = jnp.kron(i8, k_w2)
    kw3p = jnp.kron(i8, k_w3)
    kb1p = jnp.tile(k_b1, 8).reshape(1, 8 * KD)
    kb2p = jnp.tile(k_b2, 8).reshape(1, 8 * KD)
    kb3p = jnp.tile(k_b3, 8).reshape(1, 8 * C * C)

    shp = jax.ShapeDtypeStruct((N, C), f32)

    xn, x3, r0 = _tc_call(_pre_body, [shp, shp, shp], nodes, fc_w1,
                          fc_b1.reshape(1, C), fc_w2, fc_b2.reshape(1, C),
                          root0, bias0.reshape(1, C))

    fhr, fhi, gwr, gwi = consts
    ehr, ehi, dwr, dwi = iconsts
    xt = xn.reshape(4, H, W, C).transpose(1, 0, 3, 2).reshape(H * 4 * C, W)
    d12 = jax.ShapeDtypeStruct((4096, 12), f32)
    y1r, y1i = _tc_call(_specw_body, [d12, d12], xt, gwr, gwi)
    d768 = jax.ShapeDtypeStruct((24, 768), f32)
    ar, ai = _tc_call(_spech_body, [d768, d768],
                      y1r.reshape(H, 768), y1i.reshape(H, 768), fhr, fhi)
    d4 = jax.ShapeDtypeStruct((24, 4, 16, 12), f32)
    osr, osi = _tc_call(_specmix_body, [d4, d4],
                        ar.reshape(24, 4, 16, 12), ai.reshape(24, 4, 16, 12),
                        wtr, wti)
    dp = jax.ShapeDtypeStruct((H, 768), f32)
    prr, pri = _tc_call(_specih_body, [dp, dp], osr.reshape(24, 768),
                        osi.reshape(24, 768), ehr, ehi)
    z = _tc_call(_speciw_body, jax.ShapeDtypeStruct((4096, W), f32),
                 prr.reshape(4096, 12), pri.reshape(4096, 12), dwr, dwi)
    z1 = z.reshape(H, 4, C, W).transpose(1, 0, 3, 2).reshape(N, C)

    x1, _ = _tc_call(_post_body, [shp, shp], z1, conv1_w.T,
                     conv1_b.reshape(1, 2 * C), conv2_w.T,
                     conv2_b.reshape(1, C))

    eap = edge_attrs.reshape(E // 8, 8 * ED)
    CHP = CH // 8

    mlp_grid = pl.GridSpec(
        grid=(E // CH,),
        in_specs=[
            pl.BlockSpec((CHP, 8 * ED), lambda i: (i, 0)),
            pl.BlockSpec((8 * ED, 8 * KD), lambda i: (0, 0)),
            pl.BlockSpec((1, 8 * KD), lambda i: (0, 0)),
            pl.BlockSpec((8 * KD, 8 * KD), lambda i: (0, 0)),
            pl.BlockSpec((1, 8 * KD), lambda i: (0, 0)),
        ],
        out_specs=pl.BlockSpec((CHP, 8 * KD), lambda i: (i, 0)),
    )
    h2p = pl.pallas_call(
        _mlp_body, grid_spec=mlp_grid,
        out_shape=jax.ShapeDtypeStruct((E // 8, 8 * KD), f32),
    )(eap, kw1p, kb1p, kw2p, kb2p)

    def msg_pass(g):
        gath = _sc_gather(g, src)
        grid = pl.GridSpec(
            grid=(E // CH,),
            in_specs=[
                pl.BlockSpec((CHP, 8 * KD), lambda i: (i, 0)),
                pl.BlockSpec((CHP, 128), lambda i: (i, 0)),
                pl.BlockSpec((8 * KD, 8 * C * C), lambda i: (0, 0)),
                pl.BlockSpec((1, 8 * C * C), lambda i: (0, 0)),
                pl.BlockSpec((128, 8 * C * C), lambda i: (0, 0)),
                pl.BlockSpec((8 * C * C, 128), lambda i: (0, 0)),
            ],
            out_specs=pl.BlockSpec((CHP, 128), lambda i: (i, 0)),
        )
        return pl.pallas_call(
            _msg_body, grid_spec=grid,
            out_shape=jax.ShapeDtypeStruct((E // 8, 128), f32),
        )(h2p, gath, kw3p, kb3p, rm8, sm8)

    zeros = jnp.zeros((N, C), f32)

    msg0 = msg_pass(nodes).reshape(E, C)
    parts0 = _sc_scatter(msg0, dst2d, jnp.stack([r0, zeros]))
    g1, r1 = _tc_call(_combine_body, [shp, shp], parts0, root1,
                      bias1.reshape(1, C))

    msg1 = msg_pass(g1).reshape(E, C)
    parts1 = _sc_scatter(msg1, dst2d, jnp.stack([r1, zeros]))

    out = _tc_call(_final_body, jax.ShapeDtypeStruct((N, C), f32),
                   x1, parts1, x3)
    return out


# speedup vs baseline: 1.1700x; 1.1700x over previous
import functools

import jax
import jax.numpy as jnp
import numpy as np
from jax import lax
from jax.experimental import pallas as pl
from jax.experimental.pallas import tpu as pltpu
from jax.experimental.pallas import tpu_sc as plsc

H, W, C, M1, M2, KD, ED = 64, 64, 16, 12, 12, 64, 4
N = 4 * H * W
E = 262144
NW = 32
PER_W = E // NW
CH = 2048
NCH = PER_W // CH
IDX_ROWS = CH // 128


def _gelu(x):
    return 0.5 * x * (1.0 + lax.erf(x * np.float32(1.0 / np.sqrt(2.0))))


def _inorm_rows(x_ref, o_ref, b):
    for i in range(b):
        xb = x_ref[pl.ds(i * H * W, H * W), :]
        m = jnp.mean(xb, axis=0, keepdims=True)
        v = jnp.mean((xb - m) * (xb - m), axis=0, keepdims=True)
        o_ref[pl.ds(i * H * W, H * W), :] = (xb - m) * lax.rsqrt(v + 1e-5)


def _pre_body(nodes_ref, fcw1_ref, fcb1_ref, fcw2_ref, fcb2_ref, root0_ref,
              bias0_ref, xn_ref, x3_ref, r0_ref):
    _inorm_rows(nodes_ref, xn_ref, 4)
    nd = nodes_ref[...]
    h = _gelu(jnp.dot(nd, fcw1_ref[...], preferred_element_type=jnp.float32)
              + fcb1_ref[...])
    x3_ref[...] = jnp.dot(h, fcw2_ref[...], preferred_element_type=jnp.float32) \
        + fcb2_ref[...]
    r0_ref[...] = jnp.dot(nd, root0_ref[...], preferred_element_type=jnp.float32) \
        + bias0_ref[...]


def _specw_body(x_ref, gwr_ref, gwi_ref, yr_ref, yi_ref):
    x = x_ref[...]
    yr_ref[...] = jnp.dot(x, gwr_ref[...], preferred_element_type=jnp.float32)
    yi_ref[...] = jnp.dot(x, gwi_ref[...], preferred_element_type=jnp.float32)


def _spech_body(yr_ref, yi_ref, fhr_ref, fhi_ref, ar_ref, ai_ref):
    yr = yr_ref[...]
    yi = yi_ref[...]
    fhr = fhr_ref[...]
    fhi = fhi_ref[...]
    ar_ref[...] = jnp.dot(fhr, yr, preferred_element_type=jnp.float32) \
        - jnp.dot(fhi, yi, preferred_element_type=jnp.float32)
    ai_ref[...] = jnp.dot(fhr, yi, preferred_element_type=jnp.float32) \
        + jnp.dot(fhi, yr, preferred_element_type=jnp.float32)


def _specmix_body(ar_ref, ai_ref, wtr_ref, wti_ref, osr_ref, osi_ref):
    osr = jnp.zeros((24, 4, 16, 12), jnp.float32)
    osi = jnp.zeros((24, 4, 16, 12), jnp.float32)
    yvr = ar_ref[...]
    yvi = ai_ref[...]
    for i in range(16):
        br = yvr[:, :, i, :][:, :, None, :]
        bi = yvi[:, :, i, :][:, :, None, :]
        wr = wtr_ref[i][:, None, :, :]
        wi = wti_ref[i][:, None, :, :]
        osr = osr + br * wr - bi * wi
        osi = osi + br * wi + bi * wr
    osr_ref[...] = osr
    osi_ref[...] = osi


def _specih_body(osr_ref, osi_ref, ehr_ref, ehi_ref, pr_ref, pi_ref):
    osr = osr_ref[...]
    osi = osi_ref[...]
    ehr = ehr_ref[...]
    ehi = ehi_ref[...]
    pr_ref[...] = jnp.dot(ehr, osr, preferred_element_type=jnp.float32) \
        - jnp.dot(ehi, osi, preferred_element_type=jnp.float32)
    pi_ref[...] = jnp.dot(ehr, osi, preferred_element_type=jnp.float32) \
        + jnp.dot(ehi, osr, preferred_element_type=jnp.float32)


def _speciw_body(pr_ref, pi_ref, dwr_ref, dwi_ref, z_ref):
    z_ref[...] = jnp.dot(pr_ref[...], dwr_ref[...],
                         preferred_element_type=jnp.float32) \
        - jnp.dot(pi_ref[...], dwi_ref[...],
                  preferred_element_type=jnp.float32)


def _post_body(z_ref, c1w_ref, c1b_ref, c2w_ref, c2b_ref, x1_ref, zn_ref):
    _inorm_rows(z_ref, zn_ref, 4)
    h = _gelu(jnp.dot(zn_ref[...], c1w_ref[...],
                      preferred_element_type=jnp.float32) + c1b_ref[...])
    x1_ref[...] = jnp.dot(h, c2w_ref[...], preferred_element_type=jnp.float32) \
        + c2b_ref[...]


def _mlp_body(ea_ref, kw1_ref, kb1_ref, kw2_ref, kb2_ref, h2_ref):
    h1 = _gelu(jnp.dot(ea_ref[...], kw1_ref[...],
                       preferred_element_type=jnp.float32) + kb1_ref[...])
    h2_ref[...] = _gelu(jnp.dot(h1, kw2_ref[...],
                                preferred_element_type=jnp.float32)
                        + kb2_ref[...])


def _msg_body(h2_ref, g_ref, kw3_ref, kb3_ref, rm_ref, sm_ref, msg_ref):
    wf = jnp.dot(h2_ref[...], kw3_ref[...],
                 preferred_element_type=jnp.float32) + kb3_ref[...]
    xe = jnp.dot(g_ref[...], rm_ref[...], preferred_element_type=jnp.float32)
    msg_ref[...] = jnp.dot(xe * wf, sm_ref[...],
                           preferred_element_type=jnp.float32)


def _combine_body(parts_ref, root1_ref, bias1_ref, g1_ref, r1_ref):
    g1 = _gelu(parts_ref[0] + parts_ref[1])
    g1_ref[...] = g1
    r1_ref[...] = jnp.dot(g1, root1_ref[...],
                          preferred_element_type=jnp.float32) + bias1_ref[...]


def _final_body(x1_ref, parts_ref, x3_ref, out_ref):
    out_ref[...] = _gelu(x1_ref[...] + parts_ref[0] + parts_ref[1] + x3_ref[...])


def _tc_call(body, out_shapes, *args):
    return pl.pallas_call(
        body,
        out_shape=out_shapes,
    )(*args)


def _sc_gather(g, src):
    mesh = plsc.VectorSubcoreMesh(core_axis_name="c", subcore_axis_name="s")

    @functools.partial(
        pl.kernel,
        out_type=jax.ShapeDtypeStruct((E, 16), jnp.float32),
        mesh=mesh,
        compiler_params=pltpu.CompilerParams(use_tc_tiling_on_sc=False),
        scratch_types=[
            pltpu.VMEM((CH,), jnp.int32),
            pltpu.VMEM((CH, 16), jnp.float32),
            pltpu.SemaphoreType.DMA,
        ],
    )
    def k(g_hbm, src_hbm, out_hbm, idx_v, rows_v, sem):
        wid = lax.axis_index("s") * 2 + lax.axis_index("c")
        for j in range(NCH):
            base = wid * PER_W + j * CH
            pltpu.sync_copy(src_hbm.at[pl.ds(base, CH)], idx_v)
            pltpu.async_copy(g_hbm.at[idx_v], rows_v, sem).wait()
            pltpu.sync_copy(rows_v, out_hbm.at[pl.ds(base, CH)])

    return k(g, src).reshape(E // 8, 128)


def _sc_scatter(msg, dst2d, r, z):
    mesh = plsc.VectorSubcoreMesh(core_axis_name="c", subcore_axis_name="s")

    @functools.partial(
        pl.kernel,
        out_type=jax.ShapeDtypeStruct((2, N, 16), jnp.float32),
        mesh=mesh,
        compiler_params=pltpu.CompilerParams(use_tc_tiling_on_sc=False),
        scratch_types=[
            pltpu.VMEM((IDX_ROWS, 128), jnp.int32),
            pltpu.VMEM((CH, 16), jnp.float32),
            pltpu.VMEM_SHARED((N, 16), jnp.float32),
        ],
    )
    def k(msg_hbm, dst_hbm, r_hbm, z_hbm, out_hbm, idx_v, msg_v, acc_sh):
        cid = lax.axis_index("c")
        sid = lax.axis_index("s")
        wid = sid * 2 + cid

        @pl.when((sid == 0) & (cid == 0))
        def _():
            pltpu.sync_copy(r_hbm, acc_sh)

        @pl.when((sid == 0) & (cid == 1))
        def _():
            pltpu.sync_copy(z_hbm, acc_sh)

        plsc.subcore_barrier()
        for j in range(NCH):
            ebase = wid * PER_W + j * CH
            pltpu.sync_copy(dst_hbm.at[pl.ds(ebase // 128, IDX_ROWS)], idx_v)
            pltpu.sync_copy(msg_hbm.at[pl.ds(ebase, CH)], msg_v)
            for t in range(IDX_ROWS):
                pltpu.sync_copy(msg_v.at[pl.ds(t * 128, 128)],
                                acc_sh.at[idx_v.at[t]], add=True)
        plsc.subcore_barrier()
        rows = N // 16
        pltpu.sync_copy(acc_sh.at[pl.ds(sid * rows, rows)],
                        out_hbm.at[cid, pl.ds(sid * rows, rows)])

    return k(msg, dst2d, r, z)


def kernel(nodes, edge_index, edge_attrs, batchsize, w1r, w1i, w2r, w2i,
           conv1_w, conv1_b, conv2_w, conv2_b, k_w1, k_b1, k_w2, k_b2,
           k_w3, k_b3, root0, bias0, root1, bias1, fc_w1, fc_b1, fc_w2,
           fc_b2):
    f32 = jnp.float32
    src = edge_index[0]
    dst2d = edge_index[1].reshape(E // 128, 128)

    h_idx = np.arange(H)
    r_fwd = np.concatenate([np.arange(M1), np.arange(H - M1, H)])
    fh = np.exp(-2j * np.pi * np.outer(r_fwd, h_idx) / H) / H
    gw = np.exp(-2j * np.pi * np.outer(np.arange(W), np.arange(M2)) / W)
    rho = np.concatenate([np.arange(M1), np.arange(21, 33)])
    eh = np.exp(2j * np.pi * np.outer(h_idx, rho) / H)
    alpha = np.where(np.arange(M2) == 0, 1.0, 2.0)
    dw = alpha[:, None] * np.exp(
        2j * np.pi * np.outer(np.arange(M2), np.arange(W)) / W) / W
    consts = [jnp.asarray(m, f32) for m in
              (fh.real, fh.imag, gw.real, gw.imag)]
    wtr = jnp.concatenate([w1r, w2r], axis=2).transpose(0, 2, 1, 3)
    wti = jnp.concatenate([w1i, w2i], axis=2).transpose(0, 2, 1, 3)
    iconsts = [jnp.asarray(m, f32) for m in
               (eh.real, eh.imag, dw.real, dw.imag)]

    i8 = jnp.eye(8, dtype=f32)
    rm8 = jnp.asarray(np.kron(np.eye(8),
                              np.kron(np.eye(16), np.ones((1, 16)))), f32)
    sm8 = jnp.asarray(np.kron(np.eye(8), np.tile(np.eye(16), (16, 1))), f32)
    kw1p = jnp.kron(i8, k_w1)
    kw2p = jnp.kron(i8, k_w2)
    kw3p = jnp.kron(i8, k_w3)
    kb1p = jnp.tile(k_b1, 8).reshape(1, 8 * KD)
    kb2p = jnp.tile(k_b2, 8).reshape(1, 8 * KD)
    kb3p = jnp.tile(k_b3, 8).reshape(1, 8 * C * C)

    shp = jax.ShapeDtypeStruct((N, C), f32)

    xn, x3, r0 = _tc_call(_pre_body, [shp, shp, shp], nodes, fc_w1,
                          fc_b1.reshape(1, C), fc_w2, fc_b2.reshape(1, C),
                          root0, bias0.reshape(1, C))

    fhr, fhi, gwr, gwi = consts
    ehr, ehi, dwr, dwi = iconsts
    xt = xn.reshape(4, H, W, C).transpose(1, 0, 3, 2).reshape(H * 4 * C, W)
    d12 = jax.ShapeDtypeStruct((4096, 12), f32)
    y1r, y1i = _tc_call(_specw_body, [d12, d12], xt, gwr, gwi)
    d768 = jax.ShapeDtypeStruct((24, 768), f32)
    ar, ai = _tc_call(_spech_body, [d768, d768],
                      y1r.reshape(H, 768), y1i.reshape(H, 768), fhr, fhi)
    d4 = jax.ShapeDtypeStruct((24, 4, 16, 12), f32)
    osr, osi = _tc_call(_specmix_body, [d4, d4],
                        ar.reshape(24, 4, 16, 12), ai.reshape(24, 4, 16, 12),
                        wtr, wti)
    dp = jax.ShapeDtypeStruct((H, 768), f32)
    prr, pri = _tc_call(_specih_body, [dp, dp], osr.reshape(24, 768),
                        osi.reshape(24, 768), ehr, ehi)
    z = _tc_call(_speciw_body, jax.ShapeDtypeStruct((4096, W), f32),
                 prr.reshape(4096, 12), pri.reshape(4096, 12), dwr, dwi)
    z1 = z.reshape(H, 4, C, W).transpose(1, 0, 3, 2).reshape(N, C)

    x1, _ = _tc_call(_post_body, [shp, shp], z1, conv1_w.T,
                     conv1_b.reshape(1, 2 * C), conv2_w.T,
                     conv2_b.reshape(1, C))

    eap = edge_attrs.reshape(E // 8, 8 * ED)
    MCH = 8192
    CHP = MCH // 8

    mlp_grid = pl.GridSpec(
        grid=(E // MCH,),
        in_specs=[
            pl.BlockSpec((CHP, 8 * ED), lambda i: (i, 0)),
            pl.BlockSpec((8 * ED, 8 * KD), lambda i: (0, 0)),
            pl.BlockSpec((1, 8 * KD), lambda i: (0, 0)),
            pl.BlockSpec((8 * KD, 8 * KD), lambda i: (0, 0)),
            pl.BlockSpec((1, 8 * KD), lambda i: (0, 0)),
        ],
        out_specs=pl.BlockSpec((CHP, 8 * KD), lambda i: (i, 0)),
    )
    h2p = pl.pallas_call(
        _mlp_body, grid_spec=mlp_grid,
        out_shape=jax.ShapeDtypeStruct((E // 8, 8 * KD), f32),
    )(eap, kw1p, kb1p, kw2p, kb2p)

    def msg_pass(g):
        gath = _sc_gather(g, src)
        grid = pl.GridSpec(
            grid=(E // MCH,),
            in_specs=[
                pl.BlockSpec((CHP, 8 * KD), lambda i: (i, 0)),
                pl.BlockSpec((CHP, 128), lambda i: (i, 0)),
                pl.BlockSpec((8 * KD, 8 * C * C), lambda i: (0, 0)),
                pl.BlockSpec((1, 8 * C * C), lambda i: (0, 0)),
                pl.BlockSpec((128, 8 * C * C), lambda i: (0, 0)),
                pl.BlockSpec((8 * C * C, 128), lambda i: (0, 0)),
            ],
            out_specs=pl.BlockSpec((CHP, 128), lambda i: (i, 0)),
        )
        return pl.pallas_call(
            _msg_body, grid_spec=grid,
            out_shape=jax.ShapeDtypeStruct((E // 8, 128), f32),
        )(h2p, gath, kw3p, kb3p, rm8, sm8)

    zeros = jnp.zeros((N, C), f32)

    msg0 = msg_pass(nodes).reshape(E, C)
    parts0 = _sc_scatter(msg0, dst2d, r0, zeros)
    g1, r1 = _tc_call(_combine_body, [shp, shp], parts0, root1,
                      bias1.reshape(1, C))

    msg1 = msg_pass(g1).reshape(E, C)
    parts1 = _sc_scatter(msg1, dst2d, r1, zeros)

    out = _tc_call(_final_body, jax.ShapeDtypeStruct((N, C), f32),
                   x1, parts1, x3)
    return out


# speedup vs baseline: 1.2588x; 1.0759x over previous
import functools

import jax
import jax.numpy as jnp
import numpy as np
from jax import lax
from jax.experimental import pallas as pl
from jax.experimental.pallas import tpu as pltpu
from jax.experimental.pallas import tpu_sc as plsc

H, W, C, M1, M2, KD, ED = 64, 64, 16, 12, 12, 64, 4
N = 4 * H * W
E = 262144
NW = 32
PER_W = E // NW
CH = 2048
NCH = PER_W // CH
IDX_ROWS = CH // 128


def _gelu(x):
    return 0.5 * x * (1.0 + lax.erf(x * np.float32(1.0 / np.sqrt(2.0))))


def _inorm_rows(x_ref, o_ref, b):
    for i in range(b):
        xb = x_ref[pl.ds(i * H * W, H * W), :]
        m = jnp.mean(xb, axis=0, keepdims=True)
        v = jnp.mean((xb - m) * (xb - m), axis=0, keepdims=True)
        o_ref[pl.ds(i * H * W, H * W), :] = (xb - m) * lax.rsqrt(v + 1e-5)


def _pre_body(nodes_ref, fcw1_ref, fcb1_ref, fcw2_ref, fcb2_ref, root0_ref,
              bias0_ref, xn_ref, x3_ref, r0_ref):
    _inorm_rows(nodes_ref, xn_ref, 4)
    nd = nodes_ref[...]
    h = _gelu(jnp.dot(nd, fcw1_ref[...], preferred_element_type=jnp.float32)
              + fcb1_ref[...])
    x3_ref[...] = jnp.dot(h, fcw2_ref[...], preferred_element_type=jnp.float32) \
        + fcb2_ref[...]
    r0_ref[...] = jnp.dot(nd, root0_ref[...], preferred_element_type=jnp.float32) \
        + bias0_ref[...]


def _specw_body(x_ref, gwr_ref, gwi_ref, yr_ref, yi_ref):
    x = x_ref[...]
    yr_ref[...] = jnp.dot(x, gwr_ref[...], preferred_element_type=jnp.float32)
    yi_ref[...] = jnp.dot(x, gwi_ref[...], preferred_element_type=jnp.float32)


def _spech_body(yr_ref, yi_ref, fhr_ref, fhi_ref, ar_ref, ai_ref):
    yr = yr_ref[...]
    yi = yi_ref[...]
    fhr = fhr_ref[...]
    fhi = fhi_ref[...]
    ar_ref[...] = jnp.dot(fhr, yr, preferred_element_type=jnp.float32) \
        - jnp.dot(fhi, yi, preferred_element_type=jnp.float32)
    ai_ref[...] = jnp.dot(fhr, yi, preferred_element_type=jnp.float32) \
        + jnp.dot(fhi, yr, preferred_element_type=jnp.float32)


def _specmix_body(ar_ref, ai_ref, wtr_ref, wti_ref, osr_ref, osi_ref):
    osr = jnp.zeros((24, 4, 16, 12), jnp.float32)
    osi = jnp.zeros((24, 4, 16, 12), jnp.float32)
    yvr = ar_ref[...]
    yvi = ai_ref[...]
    for i in range(16):
        br = yvr[:, :, i, :][:, :, None, :]
        bi = yvi[:, :, i, :][:, :, None, :]
        wr = wtr_ref[i][:, None, :, :]
        wi = wti_ref[i][:, None, :, :]
        osr = osr + br * wr - bi * wi
        osi = osi + br * wi + bi * wr
    osr_ref[...] = osr
    osi_ref[...] = osi


def _specih_body(osr_ref, osi_ref, ehr_ref, ehi_ref, pr_ref, pi_ref):
    osr = osr_ref[...]
    osi = osi_ref[...]
    ehr = ehr_ref[...]
    ehi = ehi_ref[...]
    pr_ref[...] = jnp.dot(ehr, osr, preferred_element_type=jnp.float32) \
        - jnp.dot(ehi, osi, preferred_element_type=jnp.float32)
    pi_ref[...] = jnp.dot(ehr, osi, preferred_element_type=jnp.float32) \
        + jnp.dot(ehi, osr, preferred_element_type=jnp.float32)


def _speciw_body(pr_ref, pi_ref, dwr_ref, dwi_ref, z_ref):
    z_ref[...] = jnp.dot(pr_ref[...], dwr_ref[...],
                         preferred_element_type=jnp.float32) \
        - jnp.dot(pi_ref[...], dwi_ref[...],
                  preferred_element_type=jnp.float32)


def _post_body(z_ref, c1w_ref, c1b_ref, c2w_ref, c2b_ref, x1_ref, zn_ref):
    _inorm_rows(z_ref, zn_ref, 4)
    h = _gelu(jnp.dot(zn_ref[...], c1w_ref[...],
                      preferred_element_type=jnp.float32) + c1b_ref[...])
    x1_ref[...] = jnp.dot(h, c2w_ref[...], preferred_element_type=jnp.float32) \
        + c2b_ref[...]


def _mlp_body(eat_ref, kw1_ref, kb1_ref, kw2_ref, kb2_ref, h2_ref):
    h1 = _gelu(lax.dot_general(eat_ref[...], kw1_ref[...],
                               (((0,), (0,)), ((), ())),
                               preferred_element_type=jnp.float32)
               + kb1_ref[...])
    h2 = _gelu(jnp.dot(h1, kw2_ref[...],
                       preferred_element_type=jnp.float32) + kb2_ref[...])
    q = h2.shape[0] // 8
    h2_ref[...] = jnp.concatenate(
        [h2[j * q:(j + 1) * q, :] for j in range(8)], axis=1)


def _msg_body(h2_ref, g_ref, kw3_ref, kb3_ref, rm_ref, sm_ref, msg_ref):
    wf = jnp.dot(h2_ref[...], kw3_ref[...],
                 preferred_element_type=jnp.float32) + kb3_ref[...]
    xe = jnp.dot(g_ref[...], rm_ref[...], preferred_element_type=jnp.float32)
    msg_ref[...] = jnp.dot(xe * wf, sm_ref[...],
                           preferred_element_type=jnp.float32)


def _combine_body(parts_ref, root1_ref, bias1_ref, g1_ref, r1_ref):
    g1 = _gelu(parts_ref[0] + parts_ref[1])
    g1_ref[...] = g1
    r1_ref[...] = jnp.dot(g1, root1_ref[...],
                          preferred_element_type=jnp.float32) + bias1_ref[...]


def _final_body(x1_ref, parts_ref, x3_ref, out_ref):
    out_ref[...] = _gelu(x1_ref[...] + parts_ref[0] + parts_ref[1] + x3_ref[...])


def _tc_call(body, out_shapes, *args):
    return pl.pallas_call(
        body,
        out_shape=out_shapes,
    )(*args)


def _sc_gather(g, src):
    mesh = plsc.VectorSubcoreMesh(core_axis_name="c", subcore_axis_name="s")

    @functools.partial(
        pl.kernel,
        out_type=jax.ShapeDtypeStruct((E, 16), jnp.float32),
        mesh=mesh,
        compiler_params=pltpu.CompilerParams(use_tc_tiling_on_sc=False),
        scratch_types=[
            pltpu.VMEM((CH,), jnp.int32),
            pltpu.VMEM((CH, 16), jnp.float32),
            pltpu.SemaphoreType.DMA,
        ],
    )
    def k(g_hbm, src_hbm, out_hbm, idx_v, rows_v, sem):
        wid = lax.axis_index("s") * 2 + lax.axis_index("c")
        for j in range(NCH):
            base = wid * PER_W + j * CH
            pltpu.sync_copy(src_hbm.at[pl.ds(base, CH)], idx_v)
            pltpu.async_copy(g_hbm.at[idx_v], rows_v, sem).wait()
            pltpu.sync_copy(rows_v, out_hbm.at[pl.ds(base, CH)])

    return k(g, src).reshape(E // 8, 128)


def _sc_scatter(msg, dst2d, r, z):
    mesh = plsc.VectorSubcoreMesh(core_axis_name="c", subcore_axis_name="s")

    @functools.partial(
        pl.kernel,
        out_type=jax.ShapeDtypeStruct((2, N, 16), jnp.float32),
        mesh=mesh,
        compiler_params=pltpu.CompilerParams(use_tc_tiling_on_sc=False),
        scratch_types=[
            pltpu.VMEM((IDX_ROWS, 128), jnp.int32),
            pltpu.VMEM((CH, 16), jnp.float32),
            pltpu.VMEM_SHARED((N, 16), jnp.float32),
        ],
    )
    def k(msg_hbm, dst_hbm, r_hbm, z_hbm, out_hbm, idx_v, msg_v, acc_sh):
        cid = lax.axis_index("c")
        sid = lax.axis_index("s")
        wid = sid * 2 + cid

        @pl.when((sid == 0) & (cid == 0))
        def _():
            pltpu.sync_copy(r_hbm, acc_sh)

        @pl.when((sid == 0) & (cid == 1))
        def _():
            pltpu.sync_copy(z_hbm, acc_sh)

        plsc.subcore_barrier()
        for j in range(NCH):
            ebase = wid * PER_W + j * CH
            pltpu.sync_copy(dst_hbm.at[pl.ds(ebase // 128, IDX_ROWS)], idx_v)
            pltpu.sync_copy(msg_hbm.at[pl.ds(ebase, CH)], msg_v)
            for t in range(IDX_ROWS):
                pltpu.sync_copy(msg_v.at[pl.ds(t * 128, 128)],
                                acc_sh.at[idx_v.at[t]], add=True)
        plsc.subcore_barrier()
        rows = N // 16
        pltpu.sync_copy(acc_sh.at[pl.ds(sid * rows, rows)],
                        out_hbm.at[cid, pl.ds(sid * rows, rows)])

    return k(msg, dst2d, r, z)


def kernel(nodes, edge_index, edge_attrs, batchsize, w1r, w1i, w2r, w2i,
           conv1_w, conv1_b, conv2_w, conv2_b, k_w1, k_b1, k_w2, k_b2,
           k_w3, k_b3, root0, bias0, root1, bias1, fc_w1, fc_b1, fc_w2,
           fc_b2):
    f32 = jnp.float32
    MCH = 8192
    eperm = edge_index.reshape(2, E // MCH, 8, MCH // 8).transpose(0, 1, 3, 2)
    src = eperm[0].reshape(E)
    dst2d = eperm[1].reshape(E // 128, 128)

    h_idx = np.arange(H)
    r_fwd = np.concatenate([np.arange(M1), np.arange(H - M1, H)])
    fh = np.exp(-2j * np.pi * np.outer(r_fwd, h_idx) / H) / H
    gw = np.exp(-2j * np.pi * np.outer(np.arange(W), np.arange(M2)) / W)
    rho = np.concatenate([np.arange(M1), np.arange(21, 33)])
    eh = np.exp(2j * np.pi * np.outer(h_idx, rho) / H)
    alpha = np.where(np.arange(M2) == 0, 1.0, 2.0)
    dw = alpha[:, None] * np.exp(
        2j * np.pi * np.outer(np.arange(M2), np.arange(W)) / W) / W
    consts = [jnp.asarray(m, f32) for m in
              (fh.real, fh.imag, gw.real, gw.imag)]
    wtr = jnp.concatenate([w1r, w2r], axis=2).transpose(0, 2, 1, 3)
    wti = jnp.concatenate([w1i, w2i], axis=2).transpose(0, 2, 1, 3)
    iconsts = [jnp.asarray(m, f32) for m in
               (eh.real, eh.imag, dw.real, dw.imag)]

    i8 = jnp.eye(8, dtype=f32)
    rm8 = jnp.asarray(np.kron(np.eye(8),
                              np.kron(np.eye(16), np.ones((1, 16)))), f32)
    sm8 = jnp.asarray(np.kron(np.eye(8), np.tile(np.eye(16), (16, 1))), f32)
    kw3p = jnp.kron(i8, k_w3)
    kb3p = jnp.tile(k_b3, 8).reshape(1, 8 * C * C)

    shp = jax.ShapeDtypeStruct((N, C), f32)

    xn, x3, r0 = _tc_call(_pre_body, [shp, shp, shp], nodes, fc_w1,
                          fc_b1.reshape(1, C), fc_w2, fc_b2.reshape(1, C),
                          root0, bias0.reshape(1, C))

    fhr, fhi, gwr, gwi = consts
    ehr, ehi, dwr, dwi = iconsts
    xt = xn.reshape(4, H, W, C).transpose(1, 0, 3, 2).reshape(H * 4 * C, W)
    d12 = jax.ShapeDtypeStruct((4096, 12), f32)
    y1r, y1i = _tc_call(_specw_body, [d12, d12], xt, gwr, gwi)
    d768 = jax.ShapeDtypeStruct((24, 768), f32)
    ar, ai = _tc_call(_spech_body, [d768, d768],
                      y1r.reshape(H, 768), y1i.reshape(H, 768), fhr, fhi)
    d4 = jax.ShapeDtypeStruct((24, 4, 16, 12), f32)
    osr, osi = _tc_call(_specmix_body, [d4, d4],
                        ar.reshape(24, 4, 16, 12), ai.reshape(24, 4, 16, 12),
                        wtr, wti)
    dp = jax.ShapeDtypeStruct((H, 768), f32)
    prr, pri = _tc_call(_specih_body, [dp, dp], osr.reshape(24, 768),
                        osi.reshape(24, 768), ehr, ehi)
    z = _tc_call(_speciw_body, jax.ShapeDtypeStruct((4096, W), f32),
                 prr.reshape(4096, 12), pri.reshape(4096, 12), dwr, dwi)
    z1 = z.reshape(H, 4, C, W).transpose(1, 0, 3, 2).reshape(N, C)

    x1, _ = _tc_call(_post_body, [shp, shp], z1, conv1_w.T,
                     conv1_b.reshape(1, 2 * C), conv2_w.T,
                     conv2_b.reshape(1, C))

    eat = edge_attrs.T
    CHP = MCH // 8

    mlp_grid = pl.GridSpec(
        grid=(E // MCH,),
        in_specs=[
            pl.BlockSpec((ED, MCH), lambda i: (0, i)),
            pl.BlockSpec((ED, KD), lambda i: (0, 0)),
            pl.BlockSpec((1, KD), lambda i: (0, 0)),
            pl.BlockSpec((KD, KD), lambda i: (0, 0)),
            pl.BlockSpec((1, KD), lambda i: (0, 0)),
        ],
        out_specs=pl.BlockSpec((CHP, 8 * KD), lambda i: (i, 0)),
    )
    h2p = pl.pallas_call(
        _mlp_body, grid_spec=mlp_grid,
        out_shape=jax.ShapeDtypeStruct((E // 8, 8 * KD), f32),
    )(eat, k_w1, k_b1.reshape(1, KD), k_w2, k_b2.reshape(1, KD))

    def msg_pass(g):
        gath = _sc_gather(g, src)
        grid = pl.GridSpec(
            grid=(E // MCH,),
            in_specs=[
                pl.BlockSpec((CHP, 8 * KD), lambda i: (i, 0)),
                pl.BlockSpec((CHP, 128), lambda i: (i, 0)),
                pl.BlockSpec((8 * KD, 8 * C * C), lambda i: (0, 0)),
                pl.BlockSpec((1, 8 * C * C), lambda i: (0, 0)),
                pl.BlockSpec((128, 8 * C * C), lambda i: (0, 0)),
                pl.BlockSpec((8 * C * C, 128), lambda i: (0, 0)),
            ],
            out_specs=pl.BlockSpec((CHP, 128), lambda i: (i, 0)),
        )
        return pl.pallas_call(
            _msg_body, grid_spec=grid,
            out_shape=jax.ShapeDtypeStruct((E // 8, 128), f32),
        )(h2p, gath, kw3p, kb3p, rm8, sm8)

    zeros = jnp.zeros((N, C), f32)

    msg0 = msg_pass(nodes).reshape(E, C)
    parts0 = _sc_scatter(msg0, dst2d, r0, zeros)
    g1, r1 = _tc_call(_combine_body, [shp, shp], parts0, root1,
                      bias1.reshape(1, C))

    msg1 = msg_pass(g1).reshape(E, C)
    parts1 = _sc_scatter(msg1, dst2d, r1, zeros)

    out = _tc_call(_final_body, jax.ShapeDtypeStruct((N, C), f32),
                   x1, parts1, x3)
    return out


# speedup vs baseline: 1.5085x; 1.1984x over previous
import functools

import jax
import jax.numpy as jnp
import numpy as np
from jax import lax
from jax.experimental import pallas as pl
from jax.experimental.pallas import tpu as pltpu
from jax.experimental.pallas import tpu_sc as plsc

H, W, C, M1, M2, KD, ED = 64, 64, 16, 12, 12, 64, 4
N = 4 * H * W
E = 262144
NW = 32
PER_W = E // NW
CH = 2048
NCH = PER_W // CH
IDX_ROWS = CH // 128


def _gelu(x):
    return 0.5 * x * (1.0 + lax.erf(x * np.float32(1.0 / np.sqrt(2.0))))


def _inorm_rows(x_ref, o_ref, b):
    for i in range(b):
        xb = x_ref[pl.ds(i * H * W, H * W), :]
        m = jnp.mean(xb, axis=0, keepdims=True)
        v = jnp.mean((xb - m) * (xb - m), axis=0, keepdims=True)
        o_ref[pl.ds(i * H * W, H * W), :] = (xb - m) * lax.rsqrt(v + 1e-5)


def _pre_body(nodes_ref, fcw1_ref, fcb1_ref, fcw2_ref, fcb2_ref, root0_ref,
              bias0_ref, xn_ref, x3_ref, r0_ref):
    _inorm_rows(nodes_ref, xn_ref, 4)
    nd = nodes_ref[...]
    h = _gelu(jnp.dot(nd, fcw1_ref[...], preferred_element_type=jnp.float32)
              + fcb1_ref[...])
    x3_ref[...] = jnp.dot(h, fcw2_ref[...], preferred_element_type=jnp.float32) \
        + fcb2_ref[...]
    r0_ref[...] = jnp.dot(nd, root0_ref[...], preferred_element_type=jnp.float32) \
        + bias0_ref[...]


def _specw_body(x_ref, gwr_ref, gwi_ref, yr_ref, yi_ref):
    x = x_ref[...]
    yr_ref[...] = jnp.dot(x, gwr_ref[...], preferred_element_type=jnp.float32)
    yi_ref[...] = jnp.dot(x, gwi_ref[...], preferred_element_type=jnp.float32)


def _spech_body(yr_ref, yi_ref, fhr_ref, fhi_ref, ar_ref, ai_ref):
    yr = yr_ref[...]
    yi = yi_ref[...]
    fhr = fhr_ref[...]
    fhi = fhi_ref[...]
    ar_ref[...] = jnp.dot(fhr, yr, preferred_element_type=jnp.float32) \
        - jnp.dot(fhi, yi, preferred_element_type=jnp.float32)
    ai_ref[...] = jnp.dot(fhr, yi, preferred_element_type=jnp.float32) \
        + jnp.dot(fhi, yr, preferred_element_type=jnp.float32)


def _specmix_body(ar_ref, ai_ref, wtr_ref, wti_ref, osr_ref, osi_ref):
    osr = jnp.zeros((24, 4, 16, 12), jnp.float32)
    osi = jnp.zeros((24, 4, 16, 12), jnp.float32)
    yvr = ar_ref[...]
    yvi = ai_ref[...]
    for i in range(16):
        br = yvr[:, :, i, :][:, :, None, :]
        bi = yvi[:, :, i, :][:, :, None, :]
        wr = wtr_ref[i][:, None, :, :]
        wi = wti_ref[i][:, None, :, :]
        osr = osr + br * wr - bi * wi
        osi = osi + br * wi + bi * wr
    osr_ref[...] = osr
    osi_ref[...] = osi


def _specih_body(osr_ref, osi_ref, ehr_ref, ehi_ref, pr_ref, pi_ref):
    osr = osr_ref[...]
    osi = osi_ref[...]
    ehr = ehr_ref[...]
    ehi = ehi_ref[...]
    pr_ref[...] = jnp.dot(ehr, osr, preferred_element_type=jnp.float32) \
        - jnp.dot(ehi, osi, preferred_element_type=jnp.float32)
    pi_ref[...] = jnp.dot(ehr, osi, preferred_element_type=jnp.float32) \
        + jnp.dot(ehi, osr, preferred_element_type=jnp.float32)


def _speciw_body(pr_ref, pi_ref, dwr_ref, dwi_ref, z_ref):
    z_ref[...] = jnp.dot(pr_ref[...], dwr_ref[...],
                         preferred_element_type=jnp.float32) \
        - jnp.dot(pi_ref[...], dwi_ref[...],
                  preferred_element_type=jnp.float32)


def _post_body(z_ref, c1w_ref, c1b_ref, c2w_ref, c2b_ref, x1_ref, zn_ref):
    _inorm_rows(z_ref, zn_ref, 4)
    h = _gelu(jnp.dot(zn_ref[...], c1w_ref[...],
                      preferred_element_type=jnp.float32) + c1b_ref[...])
    x1_ref[...] = jnp.dot(h, c2w_ref[...], preferred_element_type=jnp.float32) \
        + c2b_ref[...]


def _mlp_body(eat_ref, kw1_ref, kb1_ref, kw2_ref, kb2_ref, h2_ref):
    h1 = _gelu(lax.dot_general(eat_ref[...], kw1_ref[...],
                               (((0,), (0,)), ((), ())),
                               preferred_element_type=jnp.float32)
               + kb1_ref[...])
    h2 = _gelu(jnp.dot(h1, kw2_ref[...],
                       preferred_element_type=jnp.float32) + kb2_ref[...])
    q = h2.shape[0] // 8
    h2_ref[...] = jnp.concatenate(
        [h2[j * q:(j + 1) * q, :] for j in range(8)], axis=1)


def _msg_body(h2_ref, g_ref, kw3_ref, kb3_ref, rm_ref, sm_ref, msg_ref):
    h2 = h2_ref[...]
    g = g_ref[...]
    kw3 = kw3_ref[...]
    kb3 = kb3_ref[...]
    rm = rm_ref[...]
    sm = sm_ref[...]
    outs = []
    for j in range(8):
        wfj = jnp.dot(h2[:, 64 * j:64 * (j + 1)], kw3,
                      preferred_element_type=jnp.float32) + kb3
        xej = jnp.dot(g[:, 16 * j:16 * (j + 1)], rm,
                      preferred_element_type=jnp.float32)
        outs.append(jnp.dot(xej * wfj, sm,
                            preferred_element_type=jnp.float32))
    msg_ref[...] = jnp.concatenate(outs, axis=1)


def _combine_body(parts_ref, root1_ref, bias1_ref, g1_ref, r1_ref):
    g1 = _gelu(parts_ref[0] + parts_ref[1])
    g1_ref[...] = g1
    r1_ref[...] = jnp.dot(g1, root1_ref[...],
                          preferred_element_type=jnp.float32) + bias1_ref[...]


def _final_body(x1_ref, parts_ref, x3_ref, out_ref):
    out_ref[...] = _gelu(x1_ref[...] + parts_ref[0] + parts_ref[1] + x3_ref[...])


def _tc_call(body, out_shapes, *args):
    return pl.pallas_call(
        body,
        out_shape=out_shapes,
    )(*args)


def _sc_gather(g, src):
    mesh = plsc.VectorSubcoreMesh(core_axis_name="c", subcore_axis_name="s")

    @functools.partial(
        pl.kernel,
        out_type=jax.ShapeDtypeStruct((E, 16), jnp.float32),
        mesh=mesh,
        compiler_params=pltpu.CompilerParams(use_tc_tiling_on_sc=False),
        scratch_types=[
            pltpu.VMEM((CH,), jnp.int32),
            pltpu.VMEM((CH, 16), jnp.float32),
            pltpu.SemaphoreType.DMA,
        ],
    )
    def k(g_hbm, src_hbm, out_hbm, idx_v, rows_v, sem):
        wid = lax.axis_index("s") * 2 + lax.axis_index("c")
        for j in range(NCH):
            base = wid * PER_W + j * CH
            pltpu.sync_copy(src_hbm.at[pl.ds(base, CH)], idx_v)
            pltpu.async_copy(g_hbm.at[idx_v], rows_v, sem).wait()
            pltpu.sync_copy(rows_v, out_hbm.at[pl.ds(base, CH)])

    return k(g, src).reshape(E // 8, 128)


def _sc_scatter(msg, dst2d, r, z):
    mesh = plsc.VectorSubcoreMesh(core_axis_name="c", subcore_axis_name="s")

    @functools.partial(
        pl.kernel,
        out_type=jax.ShapeDtypeStruct((2, N, 16), jnp.float32),
        mesh=mesh,
        compiler_params=pltpu.CompilerParams(use_tc_tiling_on_sc=False),
        scratch_types=[
            pltpu.VMEM((IDX_ROWS, 128), jnp.int32),
            pltpu.VMEM((CH, 16), jnp.float32),
            pltpu.VMEM_SHARED((N, 16), jnp.float32),
        ],
    )
    def k(msg_hbm, dst_hbm, r_hbm, z_hbm, out_hbm, idx_v, msg_v, acc_sh):
        cid = lax.axis_index("c")
        sid = lax.axis_index("s")
        wid = sid * 2 + cid

        @pl.when((sid == 0) & (cid == 0))
        def _():
            pltpu.sync_copy(r_hbm, acc_sh)

        @pl.when((sid == 0) & (cid == 1))
        def _():
            pltpu.sync_copy(z_hbm, acc_sh)

        plsc.subcore_barrier()
        for j in range(NCH):
            ebase = wid * PER_W + j * CH
            pltpu.sync_copy(dst_hbm.at[pl.ds(ebase // 128, IDX_ROWS)], idx_v)
            pltpu.sync_copy(msg_hbm.at[pl.ds(ebase, CH)], msg_v)
            for t in range(IDX_ROWS):
                pltpu.sync_copy(msg_v.at[pl.ds(t * 128, 128)],
                                acc_sh.at[idx_v.at[t]], add=True)
        plsc.subcore_barrier()
        rows = N // 16
        pltpu.sync_copy(acc_sh.at[pl.ds(sid * rows, rows)],
                        out_hbm.at[cid, pl.ds(sid * rows, rows)])

    return k(msg, dst2d, r, z)


def kernel(nodes, edge_index, edge_attrs, batchsize, w1r, w1i, w2r, w2i,
           conv1_w, conv1_b, conv2_w, conv2_b, k_w1, k_b1, k_w2, k_b2,
           k_w3, k_b3, root0, bias0, root1, bias1, fc_w1, fc_b1, fc_w2,
           fc_b2):
    f32 = jnp.float32
    MCH = 8192
    eperm = edge_index.reshape(2, E // MCH, 8, MCH // 8).transpose(0, 1, 3, 2)
    src = eperm[0].reshape(E)
    dst2d = eperm[1].reshape(E // 128, 128)

    h_idx = np.arange(H)
    r_fwd = np.concatenate([np.arange(M1), np.arange(H - M1, H)])
    fh = np.exp(-2j * np.pi * np.outer(r_fwd, h_idx) / H) / H
    gw = np.exp(-2j * np.pi * np.outer(np.arange(W), np.arange(M2)) / W)
    rho = np.concatenate([np.arange(M1), np.arange(21, 33)])
    eh = np.exp(2j * np.pi * np.outer(h_idx, rho) / H)
    alpha = np.where(np.arange(M2) == 0, 1.0, 2.0)
    dw = alpha[:, None] * np.exp(
        2j * np.pi * np.outer(np.arange(M2), np.arange(W)) / W) / W
    consts = [jnp.asarray(m, f32) for m in
              (fh.real, fh.imag, gw.real, gw.imag)]
    wtr = jnp.concatenate([w1r, w2r], axis=2).transpose(0, 2, 1, 3)
    wti = jnp.concatenate([w1i, w2i], axis=2).transpose(0, 2, 1, 3)
    iconsts = [jnp.asarray(m, f32) for m in
               (eh.real, eh.imag, dw.real, dw.imag)]

    i8 = jnp.eye(8, dtype=f32)
    rm8 = jnp.asarray(np.kron(np.eye(16), np.ones((1, 16))), f32)
    sm8 = jnp.asarray(np.tile(np.eye(16), (16, 1)), f32)
    kw3p = k_w3
    kb3p = k_b3.reshape(1, C * C)

    shp = jax.ShapeDtypeStruct((N, C), f32)

    xn, x3, r0 = _tc_call(_pre_body, [shp, shp, shp], nodes, fc_w1,
                          fc_b1.reshape(1, C), fc_w2, fc_b2.reshape(1, C),
                          root0, bias0.reshape(1, C))

    fhr, fhi, gwr, gwi = consts
    ehr, ehi, dwr, dwi = iconsts
    xt = xn.reshape(4, H, W, C).transpose(1, 0, 3, 2).reshape(H * 4 * C, W)
    d12 = jax.ShapeDtypeStruct((4096, 12), f32)
    y1r, y1i = _tc_call(_specw_body, [d12, d12], xt, gwr, gwi)
    d768 = jax.ShapeDtypeStruct((24, 768), f32)
    ar, ai = _tc_call(_spech_body, [d768, d768],
                      y1r.reshape(H, 768), y1i.reshape(H, 768), fhr, fhi)
    d4 = jax.ShapeDtypeStruct((24, 4, 16, 12), f32)
    osr, osi = _tc_call(_specmix_body, [d4, d4],
                        ar.reshape(24, 4, 16, 12), ai.reshape(24, 4, 16, 12),
                        wtr, wti)
    dp = jax.ShapeDtypeStruct((H, 768), f32)
    prr, pri = _tc_call(_specih_body, [dp, dp], osr.reshape(24, 768),
                        osi.reshape(24, 768), ehr, ehi)
    z = _tc_call(_speciw_body, jax.ShapeDtypeStruct((4096, W), f32),
                 prr.reshape(4096, 12), pri.reshape(4096, 12), dwr, dwi)
    z1 = z.reshape(H, 4, C, W).transpose(1, 0, 3, 2).reshape(N, C)

    x1, _ = _tc_call(_post_body, [shp, shp], z1, conv1_w.T,
                     conv1_b.reshape(1, 2 * C), conv2_w.T,
                     conv2_b.reshape(1, C))

    eat = edge_attrs.T
    CHP = MCH // 8

    mlp_grid = pl.GridSpec(
        grid=(E // MCH,),
        in_specs=[
            pl.BlockSpec((ED, MCH), lambda i: (0, i)),
            pl.BlockSpec((ED, KD), lambda i: (0, 0)),
            pl.BlockSpec((1, KD), lambda i: (0, 0)),
            pl.BlockSpec((KD, KD), lambda i: (0, 0)),
            pl.BlockSpec((1, KD), lambda i: (0, 0)),
        ],
        out_specs=pl.BlockSpec((CHP, 8 * KD), lambda i: (i, 0)),
    )
    h2p = pl.pallas_call(
        _mlp_body, grid_spec=mlp_grid,
        out_shape=jax.ShapeDtypeStruct((E // 8, 8 * KD), f32),
    )(eat, k_w1, k_b1.reshape(1, KD), k_w2, k_b2.reshape(1, KD))

    def msg_pass(g):
        gath = _sc_gather(g, src)
        grid = pl.GridSpec(
            grid=(E // MCH,),
            in_specs=[
                pl.BlockSpec((CHP, 8 * KD), lambda i: (i, 0)),
                pl.BlockSpec((CHP, 128), lambda i: (i, 0)),
                pl.BlockSpec((KD, C * C), lambda i: (0, 0)),
                pl.BlockSpec((1, C * C), lambda i: (0, 0)),
                pl.BlockSpec((C, C * C), lambda i: (0, 0)),
                pl.BlockSpec((C * C, C), lambda i: (0, 0)),
            ],
            out_specs=pl.BlockSpec((CHP, 128), lambda i: (i, 0)),
        )
        return pl.pallas_call(
            _msg_body, grid_spec=grid,
            out_shape=jax.ShapeDtypeStruct((E // 8, 128), f32),
        )(h2p, gath, kw3p, kb3p, rm8, sm8)

    zeros = jnp.zeros((N, C), f32)

    msg0 = msg_pass(nodes).reshape(E, C)
    parts0 = _sc_scatter(msg0, dst2d, r0, zeros)
    g1, r1 = _tc_call(_combine_body, [shp, shp], parts0, root1,
                      bias1.reshape(1, C))

    msg1 = msg_pass(g1).reshape(E, C)
    parts1 = _sc_scatter(msg1, dst2d, r1, zeros)

    out = _tc_call(_final_body, jax.ShapeDtypeStruct((N, C), f32),
                   x1, parts1, x3)
    return out
